# baseline (speedup 1.0000x reference)
import functools

import jax
import jax.numpy as jnp
from jax import lax
from jax.experimental import pallas as pl
from jax.experimental.pallas import tpu as pltpu
from jax.experimental.pallas import tpu_sc as plsc

_CLS_TOKEN_ID = 2
_CHUNK = 16


@functools.lru_cache(maxsize=None)
def _pooler(L, B, S, D):
    mesh = plsc.ScalarSubcoreMesh(axis_name="c", num_cores=1)
    n_chunks = S // _CHUNK

    @functools.partial(
        pl.kernel,
        mesh=mesh,
        compiler_params=pltpu.CompilerParams(needs_layout_passes=False),
        out_type=jax.ShapeDtypeStruct((B, D), jnp.float32),
        scratch_types=[
            pltpu.SMEM((B, _CHUNK), jnp.int32),
            pltpu.SemaphoreType.DMA,
        ],
    )
    def pool(states_hbm, ids_hbm, out_hbm, ids_s, sem):
        def state_row(b, idx):
            return (L - 1) * (B * S) + b * S + idx

        spec = []
        for b in range(B):
            cp = pltpu.make_async_copy(
                states_hbm.at[pl.ds(state_row(b, S - 1), 1)],
                out_hbm.at[pl.ds(b, 1)],
                sem,
            )
            cp.start()
            spec.append(cp)

        def cond(carry):
            i, idx = carry
            found = jnp.int32(1)
            for b in range(B):
                found = found & (idx[b] >= 0)
            return jnp.logical_and(found == 0, i >= 0)

        def body(carry):
            i, idx = carry
            for b in range(B):
                pltpu.sync_copy(
                    ids_hbm.at[b, pl.ds(i * _CHUNK, _CHUNK)], ids_s.at[b]
                )

            def scan_row(b, prev):
                def step(j, a):
                    jj = _CHUNK - 1 - j
                    hit = ids_s[b, jj] == _CLS_TOKEN_ID
                    return jnp.where(
                        jnp.logical_and(a < 0, hit), i * _CHUNK + jj, a
                    )

                return jnp.where(
                    prev < 0, lax.fori_loop(0, _CHUNK, step, jnp.int32(-1)), prev
                )

            return i - 1, tuple(scan_row(b, idx[b]) for b in range(B))

        _, idx = lax.while_loop(
            cond, body, (jnp.int32(n_chunks - 1), tuple(jnp.int32(-1) for _ in range(B)))
        )

        for cp in spec:
            cp.wait()

        for b in range(B):
            fix = jnp.where(idx[b] < 0, S - 1, idx[b])

            @pl.when(fix != S - 1)
            def _():
                pltpu.sync_copy(
                    states_hbm.at[pl.ds(state_row(b, fix), 1)],
                    out_hbm.at[pl.ds(b, 1)],
                )

    return pool


def kernel(layer_states, input_ids, return_final):
    del return_final
    L, B, S, D = layer_states.shape
    states = layer_states.reshape(L * B * S, D)
    return _pooler(L, B, S, D)(states, input_ids)


# speedup vs baseline: 1.0016x; 1.0016x over previous
import functools

import jax
import jax.numpy as jnp
from jax import lax
from jax.experimental import pallas as pl
from jax.experimental.pallas import tpu as pltpu
from jax.experimental.pallas import tpu_sc as plsc

_CLS_TOKEN_ID = 2
_CHUNK = 16


@functools.lru_cache(maxsize=None)
def _pooler(L, B, S, D):
    mesh = plsc.ScalarSubcoreMesh(axis_name="c", num_cores=1)
    n_chunks = S // _CHUNK

    @functools.partial(
        pl.kernel,
        mesh=mesh,
        compiler_params=pltpu.CompilerParams(
            needs_layout_passes=False, skip_device_barrier=True
        ),
        out_type=jax.ShapeDtypeStruct((B, D), jnp.float32),
        scratch_types=[
            pltpu.SMEM((B, _CHUNK), jnp.int32),
            pltpu.SemaphoreType.DMA,
        ],
    )
    def pool(states_hbm, ids_hbm, out_hbm, ids_s, sem):
        def state_row(b, idx):
            return (L - 1) * (B * S) + b * S + idx

        spec = []
        for b in range(B):
            cp = pltpu.make_async_copy(
                states_hbm.at[pl.ds(state_row(b, S - 1), 1)],
                out_hbm.at[pl.ds(b, 1)],
                sem,
            )
            cp.start()
            spec.append(cp)

        def cond(carry):
            i, idx = carry
            found = jnp.int32(1)
            for b in range(B):
                found = found & (idx[b] >= 0)
            return jnp.logical_and(found == 0, i >= 0)

        def body(carry):
            i, idx = carry
            for b in range(B):
                pltpu.sync_copy(
                    ids_hbm.at[b, pl.ds(i * _CHUNK, _CHUNK)], ids_s.at[b]
                )

            def scan_row(b, prev):
                def step(j, a):
                    jj = _CHUNK - 1 - j
                    hit = ids_s[b, jj] == _CLS_TOKEN_ID
                    return jnp.where(
                        jnp.logical_and(a < 0, hit), i * _CHUNK + jj, a
                    )

                return jnp.where(
                    prev < 0, lax.fori_loop(0, _CHUNK, step, jnp.int32(-1)), prev
                )

            return i - 1, tuple(scan_row(b, idx[b]) for b in range(B))

        _, idx = lax.while_loop(
            cond, body, (jnp.int32(n_chunks - 1), tuple(jnp.int32(-1) for _ in range(B)))
        )

        for cp in spec:
            cp.wait()

        for b in range(B):
            fix = jnp.where(idx[b] < 0, S - 1, idx[b])

            @pl.when(fix != S - 1)
            def _():
                pltpu.sync_copy(
                    states_hbm.at[pl.ds(state_row(b, fix), 1)],
                    out_hbm.at[pl.ds(b, 1)],
                )

    return pool


def kernel(layer_states, input_ids, return_final):
    del return_final
    L, B, S, D = layer_states.shape
    states = layer_states.reshape(L * B * S, D)
    return _pooler(L, B, S, D)(states, input_ids)


# speedup vs baseline: 1.0035x; 1.0019x over previous
import functools

import jax
import jax.numpy as jnp
from jax import lax
from jax.experimental import pallas as pl
from jax.experimental.pallas import tpu as pltpu
from jax.experimental.pallas import tpu_sc as plsc

_CLS_TOKEN_ID = 2
_CHUNK = 16


@functools.lru_cache(maxsize=None)
def _pooler(L, B, S, D):
    mesh = plsc.ScalarSubcoreMesh(axis_name="c", num_cores=1)
    n_chunks = S // _CHUNK

    @functools.partial(
        pl.kernel,
        mesh=mesh,
        compiler_params=pltpu.CompilerParams(
            needs_layout_passes=False, skip_device_barrier=True
        ),
        out_type=jax.ShapeDtypeStruct((B, D), jnp.float32),
        scratch_types=[
            pltpu.SMEM((B, _CHUNK), jnp.int32),
            pltpu.SemaphoreType.DMA,
            pltpu.SemaphoreType.DMA,
        ],
    )
    def pool(states_hbm, ids_hbm, out_hbm, ids_s, sem, sem_ids):
        def state_row(b, idx):
            return (L - 1) * (B * S) + b * S + idx

        spec = []
        for b in range(B):
            cp = pltpu.make_async_copy(
                states_hbm.at[pl.ds(state_row(b, S - 1), 1)],
                out_hbm.at[pl.ds(b, 1)],
                sem,
            )
            cp.start()
            spec.append(cp)

        def cond(carry):
            i, idx = carry
            found = jnp.int32(1)
            for b in range(B):
                found = found & (idx[b] >= 0)
            return jnp.logical_and(found == 0, i >= 0)

        def body(carry):
            i, idx = carry
            chunk_cps = [
                pltpu.make_async_copy(
                    ids_hbm.at[b, pl.ds(i * _CHUNK, _CHUNK)], ids_s.at[b], sem_ids
                )
                for b in range(B)
            ]
            for cp in chunk_cps:
                cp.start()
            for cp in chunk_cps:
                cp.wait()

            def scan_row(b, prev):
                found = jnp.int32(-1)
                for j in range(_CHUNK):
                    found = jnp.where(
                        ids_s[b, j] == _CLS_TOKEN_ID, i * _CHUNK + j, found
                    )
                return jnp.where(prev < 0, found, prev)

            return i - 1, tuple(scan_row(b, idx[b]) for b in range(B))

        _, idx = lax.while_loop(
            cond, body, (jnp.int32(n_chunks - 1), tuple(jnp.int32(-1) for _ in range(B)))
        )

        for cp in spec:
            cp.wait()

        for b in range(B):
            fix = jnp.where(idx[b] < 0, S - 1, idx[b])

            @pl.when(fix != S - 1)
            def _():
                pltpu.sync_copy(
                    states_hbm.at[pl.ds(state_row(b, fix), 1)],
                    out_hbm.at[pl.ds(b, 1)],
                )

    return pool


def kernel(layer_states, input_ids, return_final):
    del return_final
    L, B, S, D = layer_states.shape
    states = layer_states.reshape(L * B * S, D)
    return _pooler(L, B, S, D)(states, input_ids)


# speedup vs baseline: 1.0078x; 1.0043x over previous
import functools

import jax
import jax.numpy as jnp
from jax import lax
from jax.experimental import pallas as pl
from jax.experimental.pallas import tpu as pltpu
from jax.experimental.pallas import tpu_sc as plsc

_CLS_TOKEN_ID = 2
_CHUNK = 16


@functools.lru_cache(maxsize=None)
def _pooler(L, B, S, D):
    mesh = plsc.ScalarSubcoreMesh(axis_name="c", num_cores=1)
    n_chunks = S // _CHUNK

    @functools.partial(
        pl.kernel,
        mesh=mesh,
        compiler_params=pltpu.CompilerParams(needs_layout_passes=False),
        out_type=jax.ShapeDtypeStruct((B, D), jnp.float32),
        scratch_types=[
            pltpu.SMEM((B, _CHUNK), jnp.int32),
            pltpu.SemaphoreType.DMA,
            pltpu.SemaphoreType.DMA,
        ],
    )
    def pool(states_hbm, ids_hbm, out_hbm, ids_s, sem, sem_ids):
        def state_row(b, idx):
            return (L - 1) * (B * S) + b * S + idx

        spec = []
        for b in range(B):
            cp = pltpu.make_async_copy(
                states_hbm.at[pl.ds(state_row(b, S - 1), 1)],
                out_hbm.at[pl.ds(b, 1)],
                sem,
            )
            cp.start()
            spec.append(cp)

        def cond(carry):
            i, idx = carry
            found = jnp.int32(1)
            for b in range(B):
                found = found & (idx[b] >= 0)
            return jnp.logical_and(found == 0, i >= 0)

        def body(carry):
            i, idx = carry
            chunk_cps = [
                pltpu.make_async_copy(
                    ids_hbm.at[b, pl.ds(i * _CHUNK, _CHUNK)], ids_s.at[b], sem_ids
                )
                for b in range(B)
            ]
            for cp in chunk_cps:
                cp.start()
            for cp in chunk_cps:
                cp.wait()

            def scan_row(b, prev):
                found = jnp.int32(-1)
                for j in range(_CHUNK):
                    found = jnp.where(
                        ids_s[b, j] == _CLS_TOKEN_ID, i * _CHUNK + j, found
                    )
                return jnp.where(prev < 0, found, prev)

            return i - 1, tuple(scan_row(b, idx[b]) for b in range(B))

        _, idx = lax.while_loop(
            cond, body, (jnp.int32(n_chunks - 1), tuple(jnp.int32(-1) for _ in range(B)))
        )

        for cp in spec:
            cp.wait()

        for b in range(B):
            fix = jnp.where(idx[b] < 0, S - 1, idx[b])

            @pl.when(fix != S - 1)
            def _():
                pltpu.sync_copy(
                    states_hbm.at[pl.ds(state_row(b, fix), 1)],
                    out_hbm.at[pl.ds(b, 1)],
                )

    return pool


def kernel(layer_states, input_ids, return_final):
    del return_final
    L, B, S, D = layer_states.shape
    states = layer_states.reshape(L * B * S, D)
    return _pooler(L, B, S, D)(states, input_ids)
